# baseline (speedup 1.0000x reference)
import functools

import jax
import jax.numpy as jnp
from jax import lax
from jax.experimental import pallas as pl
from jax.experimental.pallas import tpu as pltpu
from jax.experimental.pallas import tpu_sc as plsc

T = 2048
C = 1024
H = 16
D = 64
M = 32768
KNN = 8
SCALE = 0.125

NEG_INF = float("-inf")


def _matmul_bias_body(x_ref, w_ref, b_ref, o_ref):
    o_ref[...] = lax.dot_general(
        x_ref[...], w_ref[...],
        dimension_numbers=(((1,), (1,)), ((), ())),
        preferred_element_type=jnp.float32,
    ) + b_ref[...]


def _matmul_bias(x, w, b, block_cols):
    n_out = w.shape[0]
    grid = (n_out // block_cols,)
    return pl.pallas_call(
        _matmul_bias_body,
        grid=grid,
        in_specs=[
            pl.BlockSpec((x.shape[0], x.shape[1]), lambda i: (0, 0)),
            pl.BlockSpec((block_cols, w.shape[1]), lambda i: (i, 0)),
            pl.BlockSpec((1, block_cols), lambda i: (0, i)),
        ],
        out_specs=pl.BlockSpec((x.shape[0], block_cols), lambda i: (0, i)),
        out_shape=jax.ShapeDtypeStruct((x.shape[0], n_out), jnp.float32),
    )(x, w, b.reshape(1, n_out))


_MBLK = 4096
_NJ = M // _MBLK


def _topk_body(k0_ref, key_ref, idx_ref, scores):
    h = pl.program_id(0)
    j = pl.program_id(1)
    k0v = k0_ref[0, 0, :]
    kb = key_ref[0]
    dot = lax.dot_general(
        kb.astype(jnp.bfloat16), k0v.astype(jnp.bfloat16),
        dimension_numbers=(((1,), (0,)), ((), ())),
        preferred_element_type=jnp.float32)
    s_sq = jnp.sum(kb * kb, axis=1)
    q_sq = jnp.sum(k0v * k0v)
    neg = -(q_sq - 2.0 * dot + s_sq)
    scores[pl.ds(j, 1), :] = neg.reshape(1, _MBLK)

    @pl.when(j == _NJ - 1)
    def _():
        s = scores[...]
        flat = (lax.broadcasted_iota(jnp.int32, (_NJ, _MBLK), 0) * _MBLK
                + lax.broadcasted_iota(jnp.int32, (_NJ, _MBLK), 1))
        big = jnp.int32(2**31 - 1)
        out = jnp.zeros((1, 1, KNN), jnp.int32)
        lane = lax.broadcasted_iota(jnp.int32, (1, 1, KNN), 2)
        for t in range(KNN):
            m = jnp.max(s)
            sel = jnp.min(jnp.where(s == m, flat, big))
            out = jnp.where(lane == t, sel + h * M, out)
            s = jnp.where(flat == sel, NEG_INF, s)
        idx_ref[...] = out


def _knn_topk(k0, key_store):
    return pl.pallas_call(
        _topk_body,
        grid=(H, _NJ),
        in_specs=[
            pl.BlockSpec((1, 1, D), lambda h, j: (h, 0, 0)),
            pl.BlockSpec((1, _MBLK, D), lambda h, j: (h, j, 0)),
        ],
        out_specs=pl.BlockSpec((1, 1, KNN), lambda h, j: (h, 0, 0)),
        out_shape=jax.ShapeDtypeStruct((H, 1, KNN), jnp.int32),
        scratch_shapes=[pltpu.VMEM((_NJ, _MBLK), jnp.float32)],
    )(k0, key_store)


def _lane_perm(v, perm):
    return lax.gather(
        v, perm[:, None],
        dimension_numbers=lax.GatherDimensionNumbers(
            offset_dims=(), collapsed_slice_dims=(0,), start_index_map=(0,)),
        slice_sizes=(1,),
        mode=lax.GatherScatterMode.PROMISE_IN_BOUNDS)


def _lane_allreduce_sum(v, lanes):
    for k in (8, 4, 2, 1):
        v = v + _lane_perm(v, lanes ^ k)
    return v


def _sc_body(keyflat, valflat, idxflat, q0, k0, v0, out_hbm,
             idx_v, keys_v, vals_v, q_v, k_v, v_v, out_v, sem_k, sem_v):
    c = lax.axis_index("c")
    s = lax.axis_index("s")

    @pl.when(c == 0)
    def _():
        h = s
        pltpu.sync_copy(idxflat.at[pl.ds(h * KNN, KNN)], idx_v)
        cp_k = pltpu.async_copy(keyflat.at[idx_v], keys_v, sem_k)
        cp_v = pltpu.async_copy(valflat.at[idx_v], vals_v, sem_v)
        pltpu.sync_copy(q0.at[pl.ds(h * D, D)], q_v)
        pltpu.sync_copy(k0.at[pl.ds(h * D, D)], k_v)
        pltpu.sync_copy(v0.at[pl.ds(h * D, D)], v_v)
        cp_k.wait()
        cp_v.wait()

        lanes = lax.iota(jnp.int32, 16)
        logits = []
        for j in range(KNN + 1):
            part = jnp.zeros((16,), jnp.float32)
            for ch in range(D // 16):
                qc = q_v[pl.ds(ch * 16, 16)]
                if j == 0:
                    kc = k_v[pl.ds(ch * 16, 16)]
                else:
                    kc = keys_v[j - 1, pl.ds(ch * 16, 16)]
                part = part + qc * kc
            logits.append(_lane_allreduce_sum(part, lanes) * SCALE)
        m = logits[0]
        for j in range(1, KNN + 1):
            m = jnp.maximum(m, logits[j])
        z = jnp.zeros((16,), jnp.float32)
        accs = [jnp.zeros((16,), jnp.float32) for _ in range(D // 16)]
        for j in range(KNN + 1):
            e = jnp.exp(logits[j] - m)
            z = z + e
            for ch in range(D // 16):
                if j == 0:
                    vc = v_v[pl.ds(ch * 16, 16)]
                else:
                    vc = vals_v[j - 1, pl.ds(ch * 16, 16)]
                accs[ch] = accs[ch] + e * vc
        for ch in range(D // 16):
            out_v[pl.ds(ch * 16, 16)] = accs[ch] / z
        pltpu.sync_copy(out_v, out_hbm.at[pl.ds(h * D, D)])


def _sc_retrieve(keyflat, valflat, idxflat, q0flat, k0flat, v0flat):
    fn = functools.partial(
        pl.kernel,
        mesh=plsc.VectorSubcoreMesh(core_axis_name="c", subcore_axis_name="s"),
        out_type=jax.ShapeDtypeStruct((H * D,), jnp.float32),
        compiler_params=pltpu.CompilerParams(use_tc_tiling_on_sc=False),
        scratch_types=[
            pltpu.VMEM((KNN,), jnp.int32),
            pltpu.VMEM((KNN, D), jnp.float32),
            pltpu.VMEM((KNN, D), jnp.float32),
            pltpu.VMEM((D,), jnp.float32),
            pltpu.VMEM((D,), jnp.float32),
            pltpu.VMEM((D,), jnp.float32),
            pltpu.VMEM((D,), jnp.float32),
            pltpu.SemaphoreType.DMA,
            pltpu.SemaphoreType.DMA,
        ],
    )(_sc_body)
    return fn(keyflat, valflat, idxflat, q0flat, k0flat, v0flat)


_BQ = 512
_BK = 512


def _flash_body(q_ref, k_ref, v_ref, vknn_ref, o_ref):
    qi = pl.program_id(1)
    q = q_ref[0]

    def body(jj, carry):
        mprev, l, acc = carry
        kb = k_ref[0, pl.ds(jj * _BK, _BK), :]
        vb = v_ref[0, pl.ds(jj * _BK, _BK), :]
        kvpos = jj * _BK + lax.broadcasted_iota(jnp.int32, (_BK, D), 0)
        vb = jnp.where(kvpos == 0, vknn_ref[0], vb)
        sc = lax.dot_general(
            q, kb, dimension_numbers=(((1,), (1,)), ((), ())),
            preferred_element_type=jnp.float32) * SCALE
        colg = jj * _BK + lax.broadcasted_iota(jnp.int32, (_BQ, _BK), 1)
        rowg = qi * _BQ + lax.broadcasted_iota(jnp.int32, (_BQ, _BK), 0)
        sc = jnp.where(colg <= rowg, sc, NEG_INF)
        mnew = jnp.maximum(mprev, jnp.max(sc, axis=1))
        p = jnp.exp(sc - mnew[:, None])
        alpha = jnp.exp(mprev - mnew)
        lnew = l * alpha + jnp.sum(p, axis=1)
        accnew = acc * alpha[:, None] + jnp.dot(
            p, vb, preferred_element_type=jnp.float32)
        return mnew, lnew, accnew

    m0 = jnp.full((_BQ,), NEG_INF, jnp.float32)
    l0 = jnp.zeros((_BQ,), jnp.float32)
    a0 = jnp.zeros((_BQ, D), jnp.float32)
    m, l, acc = lax.fori_loop(0, qi + 1, body, (m0, l0, a0))
    o_ref[0] = acc / l[:, None]


def _flash_attention(q, k, v, vknn):
    return pl.pallas_call(
        _flash_body,
        grid=(H, T // _BQ),
        in_specs=[
            pl.BlockSpec((1, _BQ, D), lambda h, i: (h, i, 0)),
            pl.BlockSpec((1, T, D), lambda h, i: (h, 0, 0)),
            pl.BlockSpec((1, T, D), lambda h, i: (h, 0, 0)),
            pl.BlockSpec((1, 1, D), lambda h, i: (h, 0, 0)),
        ],
        out_specs=pl.BlockSpec((1, _BQ, D), lambda h, i: (h, i, 0)),
        out_shape=jax.ShapeDtypeStruct((H, T, D), jnp.float32),
    )(q, k, v, vknn)


def kernel(x, W_attn, b_attn, W_proj, b_proj, keyStore, valueStore):
    x2 = x[0]
    qkv = _matmul_bias(x2, W_attn, b_attn, 512)

    q = qkv[:, :C].reshape(T, H, D).transpose(1, 0, 2)
    k = qkv[:, C:2 * C].reshape(T, H, D).transpose(1, 0, 2)
    v = qkv[:, 2 * C:].reshape(T, H, D).transpose(1, 0, 2)

    row0 = qkv[0]
    q0 = row0[:C]
    k0 = row0[C:2 * C]
    v0 = row0[2 * C:]

    idx = _knn_topk(k0.reshape(H, 1, D), keyStore)
    vknn = _sc_retrieve(
        keyStore.reshape(H * M, D), valueStore.reshape(H * M, D),
        idx.reshape(H * KNN), q0, k0, v0)
    vknn = vknn.reshape(H, 1, D)

    att_out = _flash_attention(q, k, v, vknn)
    y2 = att_out.transpose(1, 0, 2).reshape(T, C)
    y = _matmul_bias(y2, W_proj, b_proj, 512)
    return y.reshape(1, T, C)
